# baseline (speedup 1.0000x reference)
import jax
import jax.numpy as jnp
from jax.experimental import pallas as pl
from jax.experimental.pallas import tpu as pltpu

_ALPHA = 1.0
_BETA = 0.2
_LANES = 1024
_SEGS = 100000


def _fold_unfold_body(x_ref, out_ref, hist_ref):
    x = x_ref[:]
    bm, lanes = x.shape
    lane = jax.lax.broadcasted_iota(jnp.int32, x.shape, 1) % 4
    is_c = lane == 3
    inf = jnp.float32(jnp.inf)

    def one_iter(x):
        l1 = jnp.roll(x, -1, axis=1)
        l2 = jnp.roll(x, -2, axis=1)
        fv = jnp.minimum(x, jnp.minimum(l1, l2))
        x1 = jnp.where(is_c, x + _ALPHA * jnp.roll(fv, 3, axis=1), x)
        cb = jnp.where(
            lane == 0,
            jnp.roll(x1, -3, axis=1),
            jnp.where(lane == 1, jnp.roll(x1, -2, axis=1), jnp.roll(x1, -1, axis=1)),
        )
        x2 = jnp.where(is_c, x1, x1 + _BETA * cb)
        fmask = jnp.where(lane == 0, fv, inf)
        fold = jnp.min(fmask.reshape(bm, lanes // 4, 4), axis=2)
        return x2, fold

    x2, fold0 = one_iter(x)
    x4, fold1 = one_iter(x2)
    out_ref[:] = x4
    hist_ref[:] = jnp.stack([fold0, fold1])


def _run_fold_unfold(states):
    n = states.shape[0]
    flat = states.reshape(n * 4 // _LANES, _LANES)
    m = flat.shape[0]
    bm = 200
    grid = m // bm
    out_flat, hist = pl.pallas_call(
        _fold_unfold_body,
        grid=(grid,),
        in_specs=[pl.BlockSpec((bm, _LANES), lambda i: (i, 0))],
        out_specs=[
            pl.BlockSpec((bm, _LANES), lambda i: (i, 0)),
            pl.BlockSpec((2, bm, _LANES // 4), lambda i: (0, i, 0)),
        ],
        out_shape=[
            jax.ShapeDtypeStruct((m, _LANES), jnp.float32),
            jax.ShapeDtypeStruct((2, m, _LANES // 4), jnp.float32),
        ],
        compiler_params=pltpu.CompilerParams(
            dimension_semantics=("parallel",),
        ),
    )(flat)
    updated = out_flat.reshape(n, 4)
    fold_history = hist.reshape(2, n)
    return updated, fold_history


def kernel(states, batch, iterations):
    updated, fold_history = _run_fold_unfold(states)
    aggregated = jax.ops.segment_max(
        updated, batch, num_segments=_SEGS, indices_are_sorted=True
    )
    center_out = aggregated[..., 3]
    loops_out = aggregated[..., :3]
    return (updated, aggregated, center_out, loops_out, fold_history)
